# Initial kernel scaffold; baseline (speedup 1.0000x reference)
import jax
import jax.numpy as jnp
from jax.experimental import pallas as pl


def kernel(x, edge_index, nodes_mapper, subgraphs_batch, patch_pe, mask, W_pre, b_pre, W_gcn0, b_gcn0, W_mid, b_mid, W_gcn1, b_gcn1, W_pe1, b_pe1, W_pe2, b_pe2, Wt1, bt1, Wt2, bt2, Wc1, bc1, Wc2, bc2, W_head, b_head):
    raise NotImplementedError("write your pallas kernel here")


# speedup vs baseline: 1.0001x; 1.0001x over previous
import jax
import jax.numpy as jnp
from jax.experimental import pallas as pl


def _ln(x):
    mu = x.mean(-1, keepdims=True)
    var = x.var(-1, keepdims=True)
    return (x - mu) / jnp.sqrt(var + 1e-5)


def _segment_mean(data, ids, num):
    s = jax.ops.segment_sum(data, ids, num_segments=num)
    c = jax.ops.segment_sum(jnp.ones((data.shape[0],), data.dtype), ids, num_segments=num)
    return s / jnp.maximum(c, 1.0)[:, None]


def kernel(x, edge_index, nodes_mapper, subgraphs_batch, patch_pe, mask, W_pre, b_pre, W_gcn0, b_gcn0, W_mid, b_mid, W_gcn1, b_gcn1, W_pe1, b_pe1, W_pe2, b_pe2, Wt1, bt1, Wt2, bt2, Wc1, bc1, Wc2, bc2, W_head, b_head):
    N = x.shape[0]
    M = nodes_mapper.shape[0]
    NP = patch_pe.shape[0]
    B, P = mask.shape
    E = edge_index.shape[1]
    src, dst = edge_index[0], edge_index[1]
    h = jax.nn.relu(x @ W_pre + b_pre)
    xs = h[nodes_mapper]
    deg = jax.ops.segment_sum(jnp.ones((E,), jnp.float32), dst, num_segments=M) + 1.0
    norm = 1.0 / jnp.sqrt(deg[src] * deg[dst])

    def gcn(z, W, b):
        hh = z @ W + b
        agg = jax.ops.segment_sum(hh[src] * norm[:, None], dst, num_segments=M) + hh / deg[:, None]
        return jax.nn.relu(agg)

    xs = gcn(xs, W_gcn0, b_gcn0)
    pooled = _segment_mean(xs, subgraphs_batch, NP)[subgraphs_batch]
    pooled = jax.nn.relu(pooled @ W_mid + b_mid)
    xs = xs + pooled
    xs = _segment_mean(xs, nodes_mapper, N)[nodes_mapper]
    xs = gcn(xs, W_gcn1, b_gcn1)
    sub = _segment_mean(xs, subgraphs_batch, NP)
    pe = jax.nn.relu(patch_pe @ W_pe1 + b_pe1) @ W_pe2 + b_pe2
    sub = sub + pe
    z = sub.reshape(B, P, -1)
    for l in range(Wt1.shape[0]):
        y = _ln(z)
        y = jnp.swapaxes(y, 1, 2)
        y = jax.nn.gelu(y @ Wt1[l] + bt1[l]) @ Wt2[l] + bt2[l]
        z = z + jnp.swapaxes(y, 1, 2)
        y = _ln(z)
        z = z + (jax.nn.gelu(y @ Wc1[l] + bc1[l]) @ Wc2[l] + bc2[l])
    g = (z * mask[:, :, None]).sum(1) / mask.sum(1, keepdims=True)
    out = g @ W_head + b_head
    return out


# speedup vs baseline: 4.0497x; 4.0495x over previous
import functools

import jax
import jax.numpy as jnp
from jax import lax
from jax.experimental import pallas as pl
from jax.experimental.pallas import tpu as pltpu
from jax.experimental.pallas import tpu_sc as plsc

N, M, E, H, NP, B, P, PE, C, L = 10000, 20000, 320000, 128, 512, 16, 32, 8, 10, 2
NC, NS, LANES = 2, 16, 16
NW = NC * NS
M_PAD = 20480
E_PAD = 327680
N_PAD = 10240
NP_PAD = 640

_HI = jax.lax.Precision.HIGHEST


def _mesh():
    return plsc.VectorSubcoreMesh(core_axis_name="c", subcore_axis_name="s")


def _al8(x):
    return pl.multiple_of(x, 8)


def _zero_vmem_2d(ref, rows, cols):
    zero16 = jnp.zeros((LANES,), jnp.float32)

    def body(r, _):
        for k in range(cols // LANES):
            ref[r, pl.ds(k * LANES, LANES)] = zero16
        return 0

    lax.fori_loop(0, rows, body, 0)


def _fill_vmem_1d(ref, n, value):
    val16 = jnp.full((LANES,), value, jnp.float32)

    def body(r, _):
        ref[pl.ds(r * LANES, LANES)] = val16
        return 0

    lax.fori_loop(0, n // LANES, body, 0)


def _sc_counts(dst3, nm3, sb3):
    @functools.partial(
        pl.kernel,
        out_type=(
            jax.ShapeDtypeStruct((NC * M_PAD,), jnp.float32),
            jax.ShapeDtypeStruct((NC * N_PAD,), jnp.float32),
            jax.ShapeDtypeStruct((NC * NP_PAD,), jnp.float32),
        ),
        mesh=_mesh(),
        scratch_types=[
            pltpu.VMEM((80, 128), jnp.int32),
            pltpu.VMEM((5, 128), jnp.int32),
            pltpu.VMEM((128,), jnp.float32),
            pltpu.VMEM((1280,), jnp.float32),
            pltpu.VMEM_SHARED((M_PAD,), jnp.float32),
            pltpu.VMEM_SHARED((N_PAD,), jnp.float32),
            pltpu.VMEM_SHARED((NP_PAD,), jnp.float32),
        ],
    )
    def k(dst_h, nm_h, sb_h, deg_o, cnm_o, csb_o, idx_v, idx5_v, ones_v,
          zero_v, deg_a, cnm_a, csb_a):
        cid = lax.axis_index("c")
        sid = lax.axis_index("s")
        wid = cid * NS + sid
        _fill_vmem_1d(ones_v, 128, 1.0)
        _fill_vmem_1d(zero_v, 1280, 0.0)
        pltpu.sync_copy(zero_v, deg_a.at[pl.ds(_al8(sid * 1280), 1280)])
        pltpu.sync_copy(zero_v.at[pl.ds(0, 640)],
                        cnm_a.at[pl.ds(_al8(sid * 640), 640)])

        @pl.when(sid < 5)
        def _():
            pltpu.sync_copy(zero_v.at[pl.ds(0, 128)],
                            csb_a.at[pl.ds(_al8(sid * 128), 128)])

        plsc.subcore_barrier()
        pltpu.sync_copy(dst_h.at[wid], idx_v)

        def dbody(j, _):
            pltpu.sync_copy(ones_v, deg_a.at[idx_v.at[j]], add=True)
            return 0

        lax.fori_loop(0, 80, dbody, 0)
        pltpu.sync_copy(nm_h.at[wid], idx5_v)

        def nbody(j, _):
            pltpu.sync_copy(ones_v, cnm_a.at[idx5_v.at[j]], add=True)
            return 0

        lax.fori_loop(0, 5, nbody, 0)
        pltpu.sync_copy(sb_h.at[wid], idx5_v)

        def sbody(j, _):
            pltpu.sync_copy(ones_v, csb_a.at[idx5_v.at[j]], add=True)
            return 0

        lax.fori_loop(0, 5, sbody, 0)
        plsc.subcore_barrier()
        pltpu.sync_copy(deg_a.at[pl.ds(_al8(sid * 1280), 1280)],
                        deg_o.at[pl.ds(_al8(cid * M_PAD + sid * 1280), 1280)])
        pltpu.sync_copy(cnm_a.at[pl.ds(_al8(sid * 640), 640)],
                        cnm_o.at[pl.ds(_al8(cid * N_PAD + sid * 640), 640)])

        @pl.when(sid < 5)
        def _():
            pltpu.sync_copy(
                csb_a.at[pl.ds(_al8(sid * 128), 128)],
                csb_o.at[pl.ds(_al8(cid * NP_PAD + sid * 128), 128)])

    deg_f, cnm_f, csb_f = k(dst3, nm3, sb3)
    return (deg_f.reshape(NC, M_PAD), cnm_f.reshape(NC, N_PAD),
            csb_f.reshape(NC, NP_PAD))


def _sc_gather(tab, idx3):
    rpw = idx3.shape[1]

    @functools.partial(
        pl.kernel,
        out_type=jax.ShapeDtypeStruct((NW * rpw * 128, 128), jnp.float32),
        mesh=_mesh(),
        scratch_types=[
            pltpu.VMEM((rpw, 128), jnp.int32),
            pltpu.VMEM((128, 128), jnp.float32),
            pltpu.VMEM((128, 128), jnp.float32),
            pltpu.SemaphoreType.DMA,
            pltpu.SemaphoreType.DMA,
        ],
    )
    def k(tab_h, idx_h, out_h, idx_v, rows_a, rows_b, sem_a, sem_b):
        cid = lax.axis_index("c")
        sid = lax.axis_index("s")
        wid = cid * NS + sid
        base = wid * rpw
        pltpu.sync_copy(idx_h.at[wid], idx_v)
        bufs = (rows_a, rows_b)
        sems = (sem_a, sem_b)
        descs = [None, None]
        descs[0] = pltpu.async_copy(tab_h.at[idx_v.at[0]], rows_a, sem_a)
        for j in range(rpw):
            cur = j % 2
            nxt = (j + 1) % 2
            if j + 1 < rpw:
                descs[nxt] = pltpu.async_copy(tab_h.at[idx_v.at[j + 1]],
                                              bufs[nxt], sems[nxt])
            descs[cur].wait()
            pltpu.sync_copy(bufs[cur],
                            out_h.at[pl.ds(_al8((base + j) * 128), 128)])

    return k(tab, idx3)


HALF_M = M_PAD // 2
ACC_M = 10496
IBLK = 32


def _sc_edge_segsum(hn, srcE, dstE):
    rpt = E_PAD // 128 // NS
    nblk = rpt // IBLK

    @functools.partial(
        pl.kernel,
        out_type=jax.ShapeDtypeStruct((M_PAD, H), jnp.float32),
        mesh=_mesh(),
        scratch_types=[
            pltpu.VMEM((IBLK, 128), jnp.int32),
            pltpu.VMEM((IBLK, 128), jnp.int32),
            pltpu.VMEM((128, 128), jnp.float32),
            pltpu.VMEM((128, 128), jnp.float32),
            pltpu.VMEM_SHARED((ACC_M, 128), jnp.float32),
            pltpu.SemaphoreType.DMA,
        ],
    )
    def k(tab_h, src_h, dst_h, out_h, src_v, dst_v, rows_v, zrow_v, acc, sem):
        cid = lax.axis_index("c")
        sid = lax.axis_index("s")
        _zero_vmem_2d(zrow_v, 128, 128)

        def zbody(t, _):
            pltpu.sync_copy(zrow_v,
                            acc.at[pl.ds(_al8((sid * 5 + t) * 128), 128)])
            return 0

        lax.fori_loop(0, 5, zbody, 0)

        @pl.when(sid < 2)
        def _():
            pltpu.sync_copy(zrow_v, acc.at[pl.ds(_al8((80 + sid) * 128), 128)])

        base = cid * HALF_M
        plsc.subcore_barrier()

        def blk(bi, _):
            pltpu.sync_copy(
                src_h.at[sid, pl.ds(_al8(bi * IBLK), IBLK)], src_v)
            pltpu.sync_copy(
                dst_h.at[sid, pl.ds(_al8(bi * IBLK), IBLK)], dst_v)

            def tbody(j, _):
                for kk in range(8):
                    v = dst_v[j, pl.ds(kk * LANES, LANES)] - base
                    v = jnp.where((v < 0) | (v >= HALF_M), HALF_M, v)
                    dst_v[j, pl.ds(kk * LANES, LANES)] = v
                return 0

            lax.fori_loop(0, IBLK, tbody, 0)

            def body(j, _):
                pltpu.async_copy(tab_h.at[src_v.at[j]], rows_v, sem).wait()
                pltpu.sync_copy(rows_v, acc.at[dst_v.at[j]], add=True)
                return 0

            lax.fori_loop(0, IBLK, body, 0)
            return 0

        lax.fori_loop(0, nblk, blk, 0)
        plsc.subcore_barrier()

        def obody(t, _):
            r0 = _al8((sid * 5 + t) * 128)
            pltpu.sync_copy(acc.at[pl.ds(r0, 128)],
                            out_h.at[pl.ds(_al8(cid * HALF_M + r0), 128)])
            return 0

        lax.fori_loop(0, 5, obody, 0)

    return k(hn, srcE, dstE)


def _sc_patch_segsum(data, sb3):
    @functools.partial(
        pl.kernel,
        out_type=jax.ShapeDtypeStruct((NC, NP_PAD, 128), jnp.float32),
        mesh=_mesh(),
        scratch_types=[
            pltpu.VMEM((5, 128), jnp.int32),
            pltpu.VMEM((128, 128), jnp.float32),
            pltpu.VMEM((128, 128), jnp.float32),
            pltpu.VMEM_SHARED((NP_PAD, 128), jnp.float32),
        ],
    )
    def k(data_h, sb_h, out_h, idx_v, data_v, zrow_v, acc):
        cid = lax.axis_index("c")
        sid = lax.axis_index("s")
        wid = cid * NS + sid
        _zero_vmem_2d(zrow_v, 128, 128)
        pltpu.sync_copy(zrow_v.at[pl.ds(0, 40)],
                        acc.at[pl.ds(_al8(sid * 40), 40)])
        pltpu.sync_copy(sb_h.at[wid], idx_v)
        plsc.subcore_barrier()

        def body(j, _):
            pltpu.sync_copy(
                data_h.at[pl.ds(_al8((wid * 5 + j) * 128), 128)], data_v)
            pltpu.sync_copy(data_v, acc.at[idx_v.at[j]], add=True)
            return 0

        lax.fori_loop(0, 5, body, 0)
        plsc.subcore_barrier()
        pltpu.sync_copy(acc.at[pl.ds(_al8(sid * 40), 40)],
                        out_h.at[cid, pl.ds(_al8(sid * 40), 40)])

    return k(data, sb3)


def _sc_node_segsum(data, nm3):
    @functools.partial(
        pl.kernel,
        out_type=jax.ShapeDtypeStruct((NC, N_PAD, H), jnp.float32),
        mesh=_mesh(),
        scratch_types=[
            pltpu.VMEM((5, 128), jnp.int32),
            pltpu.VMEM((128, 128), jnp.float32),
            pltpu.VMEM((128, 128), jnp.float32),
            pltpu.VMEM_SHARED((N_PAD, H), jnp.float32),
        ],
    )
    def k(data_h, nm_h, out_h, idx_v, data_v, zrow_v, acc):
        cid = lax.axis_index("c")
        sid = lax.axis_index("s")
        wid = cid * NS + sid
        _zero_vmem_2d(zrow_v, 128, 128)

        def zbody(t, _):
            pltpu.sync_copy(zrow_v,
                            acc.at[pl.ds(_al8((sid * 5 + t) * 128), 128)])
            return 0

        lax.fori_loop(0, 5, zbody, 0)
        pltpu.sync_copy(nm_h.at[wid], idx_v)
        plsc.subcore_barrier()

        def body(j, _):
            pltpu.sync_copy(
                data_h.at[pl.ds(_al8((wid * 5 + j) * 128), 128)], data_v)
            pltpu.sync_copy(data_v, acc.at[idx_v.at[j]], add=True)
            return 0

        lax.fori_loop(0, 5, body, 0)
        plsc.subcore_barrier()

        def obody(t, _):
            r0 = _al8((sid * 5 + t) * 128)
            pltpu.sync_copy(acc.at[pl.ds(r0, 128)],
                            out_h.at[cid, pl.ds(r0, 128)])
            return 0

        lax.fori_loop(0, 5, obody, 0)

    return k(data, nm3)


def _tc_pre(x, W, b):
    def body(x_r, w_r, b_r, o_r):
        o_r[...] = jnp.maximum(
            jax.lax.dot_general(x_r[...], w_r[...], (((1,), (0,)), ((), ())),
                                precision=_HI) + b_r[...], 0.0)

    return pl.pallas_call(
        body,
        out_shape=jax.ShapeDtypeStruct((N, H), jnp.float32),
        grid=(5,),
        in_specs=[
            pl.BlockSpec((2000, H), lambda i: (i, 0)),
            pl.BlockSpec((H, H), lambda i: (0, 0)),
            pl.BlockSpec((1, H), lambda i: (0, 0)),
        ],
        out_specs=pl.BlockSpec((2000, H), lambda i: (i, 0)),
    )(x, W, b)


def _tc_gcn_lin(xs, W, b, degT):
    def body(x_r, w_r, b_r, d_r, u_r, hn_r):
        deg = d_r[:, 0:1] + d_r[:, 1:2] + 1.0
        isd = lax.rsqrt(deg)
        hh = jax.lax.dot_general(x_r[...], w_r[...], (((1,), (0,)), ((), ())),
                                 precision=_HI) + b_r[...]
        u_r[...] = hh / deg
        hn_r[...] = hh * isd

    return pl.pallas_call(
        body,
        out_shape=(
            jax.ShapeDtypeStruct((M_PAD, H), jnp.float32),
            jax.ShapeDtypeStruct((M_PAD, H), jnp.float32),
        ),
        grid=(10,),
        in_specs=[
            pl.BlockSpec((2048, H), lambda i: (i, 0)),
            pl.BlockSpec((H, H), lambda i: (0, 0)),
            pl.BlockSpec((1, H), lambda i: (0, 0)),
            pl.BlockSpec((2048, 2), lambda i: (i, 0)),
        ],
        out_specs=(
            pl.BlockSpec((2048, H), lambda i: (i, 0)),
            pl.BlockSpec((2048, H), lambda i: (i, 0)),
        ),
    )(xs, W, b, degT)


def _tc_gcn_combine(tmp, u, degT):
    def body(t_r, u_r, d_r, o_r):
        deg = d_r[:, 0:1] + d_r[:, 1:2] + 1.0
        isd = lax.rsqrt(deg)
        o_r[...] = jnp.maximum(t_r[...] * isd + u_r[...], 0.0)

    return pl.pallas_call(
        body,
        out_shape=jax.ShapeDtypeStruct((M_PAD, H), jnp.float32),
        grid=(10,),
        in_specs=[
            pl.BlockSpec((2048, H), lambda i: (i, 0)),
            pl.BlockSpec((2048, H), lambda i: (i, 0)),
            pl.BlockSpec((2048, 2), lambda i: (i, 0)),
        ],
        out_specs=pl.BlockSpec((2048, H), lambda i: (i, 0)),
    )(tmp, u, degT)


def _tc_patch_mlp(p0, p1, cntT, W, b):
    def body(p0_r, p1_r, c_r, w_r, b_r, o_r):
        cnt = jnp.maximum(c_r[:, 0:1] + c_r[:, 1:2], 1.0)
        pm = (p0_r[...] + p1_r[...]) / cnt
        o_r[...] = jnp.maximum(
            jax.lax.dot_general(pm, w_r[...], (((1,), (0,)), ((), ())),
                                precision=_HI) + b_r[...], 0.0)

    return pl.pallas_call(
        body,
        out_shape=jax.ShapeDtypeStruct((NP_PAD, H), jnp.float32),
        in_specs=[
            pl.BlockSpec((NP_PAD, H), lambda: (0, 0)),
            pl.BlockSpec((NP_PAD, H), lambda: (0, 0)),
            pl.BlockSpec((NP_PAD, 2), lambda: (0, 0)),
            pl.BlockSpec((H, H), lambda: (0, 0)),
            pl.BlockSpec((1, H), lambda: (0, 0)),
        ],
        out_specs=pl.BlockSpec((NP_PAD, H), lambda: (0, 0)),
    )(p0, p1, cntT, W, b)


def _tc_add(xs1, g1):
    def body(a_r, b_r, o_r):
        o_r[...] = a_r[...] + b_r[...]

    return pl.pallas_call(
        body,
        out_shape=jax.ShapeDtypeStruct((M_PAD, H), jnp.float32),
        grid=(10,),
        in_specs=[
            pl.BlockSpec((2048, H), lambda i: (i, 0)),
            pl.BlockSpec((2048, H), lambda i: (i, 0)),
        ],
        out_specs=pl.BlockSpec((2048, H), lambda i: (i, 0)),
    )(xs1, g1)


def _tc_node_mean(s0, s1, cntT):
    def body(a_r, b_r, c_r, o_r):
        cnt = jnp.maximum(c_r[:, 0:1] + c_r[:, 1:2], 1.0)
        o_r[...] = (a_r[...] + b_r[...]) / cnt

    return pl.pallas_call(
        body,
        out_shape=jax.ShapeDtypeStruct((N_PAD, H), jnp.float32),
        grid=(5,),
        in_specs=[
            pl.BlockSpec((2048, H), lambda i: (i, 0)),
            pl.BlockSpec((2048, H), lambda i: (i, 0)),
            pl.BlockSpec((2048, 2), lambda i: (i, 0)),
        ],
        out_specs=pl.BlockSpec((2048, H), lambda i: (i, 0)),
    )(s0, s1, cntT)


def _ln(z):
    mu = z.mean(-1, keepdims=True)
    var = z.var(-1, keepdims=True)
    return (z - mu) / jnp.sqrt(var + 1e-5)


def _tc_mixer_head(p0, p1, cntT, patch_pe, W_pe1, b_pe1, W_pe2, b_pe2,
                   Wt1T, bt1c, Wt2T, bt2c, Wc1, bc1r, Wc2, bc2r, mask,
                   W_head, b_head):
    def body(p0_r, p1_r, c_r, pe_r, wp1_r, bp1_r, wp2_r, bp2_r, wt1_r, bt1_r,
             wt2_r, bt2_r, wc1_r, bc1_r, wc2_r, bc2_r, m_r, wh_r, bh_r, o_r):
        cnt = jnp.maximum(c_r[:, 0:1] + c_r[:, 1:2], 1.0)
        sub = (p0_r[...] + p1_r[...]) / cnt
        pe = jax.lax.dot_general(
            jnp.maximum(jax.lax.dot_general(pe_r[...], wp1_r[...],
                                            (((1,), (0,)), ((), ())),
                                            precision=_HI) + bp1_r[...], 0.0),
            wp2_r[...], (((1,), (0,)), ((), ())), precision=_HI) + bp2_r[...]
        z = sub + pe
        wt1 = wt1_r[...]
        bt1 = bt1_r[...]
        wt2 = wt2_r[...]
        bt2 = bt2_r[...]
        wc1 = wc1_r[...]
        bc1 = bc1_r[...]
        wc2 = wc2_r[...]
        bc2 = bc2_r[...]
        for l in range(L):
            y = _ln(z)
            wt1l = wt1[l * P:(l + 1) * P, :]
            wt2l = wt2[l * P:(l + 1) * P, :]
            bt1l = bt1[l * P:(l + 1) * P, :]
            bt2l = bt2[l * P:(l + 1) * P, :]
            touts = []
            for b in range(B):
                yb = y[b * P:(b + 1) * P, :]
                a = jax.lax.dot_general(wt1l, yb, (((1,), (0,)), ((), ())),
                                        precision=_HI) + bt1l
                a = jax.nn.gelu(a)
                a = jax.lax.dot_general(wt2l, a, (((1,), (0,)), ((), ())),
                                        precision=_HI) + bt2l
                touts.append(a)
            z = z + jnp.concatenate(touts, axis=0)
            y = _ln(z)
            cmid = jax.nn.gelu(
                jax.lax.dot_general(y, wc1[l * H:(l + 1) * H, :],
                                    (((1,), (0,)), ((), ())),
                                    precision=_HI) + bc1[l:l + 1, :])
            z = z + jax.lax.dot_general(cmid, wc2[l * 2 * H:(l + 1) * 2 * H, :],
                                        (((1,), (0,)), ((), ())),
                                        precision=_HI) + bc2[l:l + 1, :]
        m = m_r[...]
        gs = []
        for b in range(B):
            mb = m[b:b + 1, :]
            zb = z[b * P:(b + 1) * P, :]
            gb = jax.lax.dot_general(mb, zb, (((1,), (0,)), ((), ())),
                                     precision=_HI)
            gs.append(gb / jnp.sum(mb, keepdims=True))
        g = jnp.concatenate(gs, axis=0)
        o_r[...] = jax.lax.dot_general(g, wh_r[...], (((1,), (0,)), ((), ())),
                                       precision=_HI) + bh_r[...]

    return pl.pallas_call(
        body,
        out_shape=jax.ShapeDtypeStruct((B, C), jnp.float32),
        in_specs=[
            pl.BlockSpec((NP, H), lambda: (0, 0)),
            pl.BlockSpec((NP, H), lambda: (0, 0)),
            pl.BlockSpec((NP, 2), lambda: (0, 0)),
            pl.BlockSpec((NP, PE), lambda: (0, 0)),
            pl.BlockSpec((PE, H), lambda: (0, 0)),
            pl.BlockSpec((1, H), lambda: (0, 0)),
            pl.BlockSpec((H, H), lambda: (0, 0)),
            pl.BlockSpec((1, H), lambda: (0, 0)),
            pl.BlockSpec((L * P, P), lambda: (0, 0)),
            pl.BlockSpec((L * P, 1), lambda: (0, 0)),
            pl.BlockSpec((L * P, P), lambda: (0, 0)),
            pl.BlockSpec((L * P, 1), lambda: (0, 0)),
            pl.BlockSpec((L * H, 2 * H), lambda: (0, 0)),
            pl.BlockSpec((L, 2 * H), lambda: (0, 0)),
            pl.BlockSpec((L * 2 * H, H), lambda: (0, 0)),
            pl.BlockSpec((L, H), lambda: (0, 0)),
            pl.BlockSpec((B, P), lambda: (0, 0)),
            pl.BlockSpec((H, C), lambda: (0, 0)),
            pl.BlockSpec((1, C), lambda: (0, 0)),
        ],
        out_specs=pl.BlockSpec((B, C), lambda: (0, 0)),
    )(p0, p1, cntT, patch_pe, W_pe1, b_pe1, W_pe2, b_pe2,
      Wt1T, bt1c, Wt2T, bt2c, Wc1, bc1r, Wc2, bc2r, mask, W_head, b_head)


def _pad_idx(a, total, fill, lead):
    a = a.astype(jnp.int32)
    return jnp.concatenate(
        [a, jnp.full((total - a.shape[0],), fill, jnp.int32)]
    ).reshape(lead, -1, 128)


def kernel(x, edge_index, nodes_mapper, subgraphs_batch, patch_pe, mask,
           W_pre, b_pre, W_gcn0, b_gcn0, W_mid, b_mid, W_gcn1, b_gcn1,
           W_pe1, b_pe1, W_pe2, b_pe2, Wt1, bt1, Wt2, bt2, Wc1, bc1, Wc2, bc2,
           W_head, b_head):
    srcE = _pad_idx(edge_index[0], E_PAD, 0, NS)
    dstE = _pad_idx(edge_index[1], E_PAD, M, NS)
    dst3 = _pad_idx(edge_index[1], E_PAD, M, NW)
    nmG = _pad_idx(nodes_mapper, M_PAD, 0, NW)
    nmS = _pad_idx(nodes_mapper, M_PAD, N, NW)
    sbR = _pad_idx(subgraphs_batch, M_PAD, NP, NW)

    b_pre2 = b_pre.reshape(1, H)
    b_gcn0r = b_gcn0.reshape(1, H)
    b_gcn1r = b_gcn1.reshape(1, H)
    b_midr = b_mid.reshape(1, H)
    b_pe1r = b_pe1.reshape(1, H)
    b_pe2r = b_pe2.reshape(1, H)
    Wt1T = jnp.swapaxes(Wt1, 1, 2).reshape(L * P, P)
    Wt2T = jnp.swapaxes(Wt2, 1, 2).reshape(L * P, P)
    bt1c = bt1.reshape(L * P, 1)
    bt2c = bt2.reshape(L * P, 1)
    Wc1r = Wc1.reshape(L * H, 2 * H)
    Wc2r = Wc2.reshape(L * 2 * H, H)
    b_headr = b_head.reshape(1, C)

    h = _tc_pre(x, W_pre, b_pre2)
    deg_p, cnm_p, csb_p = _sc_counts(dst3, nmS, sbR)
    degT = jnp.swapaxes(deg_p, 0, 1)
    cnmT = jnp.swapaxes(cnm_p, 0, 1)
    csbT = jnp.swapaxes(csb_p, 0, 1)
    xs0 = _sc_gather(h, nmG)
    u0, hn0 = _tc_gcn_lin(xs0, W_gcn0, b_gcn0r, degT)
    tmp0 = _sc_edge_segsum(hn0, srcE, dstE)
    xs1 = _tc_gcn_combine(tmp0, u0, degT)
    pool1 = _sc_patch_segsum(xs1, sbR)
    q = _tc_patch_mlp(pool1[0], pool1[1], csbT, W_mid, b_midr)
    g1 = _sc_gather(q, sbR)
    xs2 = _tc_add(xs1, g1)
    s = _sc_node_segsum(xs2, nmS)
    meantab = _tc_node_mean(s[0], s[1], cnmT)
    xs4 = _sc_gather(meantab, nmS)
    u1, hn1 = _tc_gcn_lin(xs4, W_gcn1, b_gcn1r, degT)
    tmp1 = _sc_edge_segsum(hn1, srcE, dstE)
    xs5 = _tc_gcn_combine(tmp1, u1, degT)
    pool2 = _sc_patch_segsum(xs5, sbR)
    out = _tc_mixer_head(pool2[0, :NP], pool2[1, :NP], csbT[:NP], patch_pe,
                         W_pe1, b_pe1r, W_pe2, b_pe2r, Wt1T, bt1c, Wt2T, bt2c,
                         Wc1r, bc1, Wc2r, bc2, mask, W_head, b_headr)
    return out


# speedup vs baseline: 4.2374x; 1.0463x over previous
import functools

import jax
import jax.numpy as jnp
from jax import lax
from jax.experimental import pallas as pl
from jax.experimental.pallas import tpu as pltpu
from jax.experimental.pallas import tpu_sc as plsc

N, M, E, H, NP, B, P, PE, C, L = 10000, 20000, 320000, 128, 512, 16, 32, 8, 10, 2
NC, NS, LANES = 2, 16, 16
NW = NC * NS
M_PAD = 20480
E_PAD = 327680
N_PAD = 10240
NP_PAD = 640

_HI = jax.lax.Precision.HIGHEST


def _mesh():
    return plsc.VectorSubcoreMesh(core_axis_name="c", subcore_axis_name="s")


def _al8(x):
    return pl.multiple_of(x, 8)


def _zero_vmem_2d(ref, rows, cols):
    zero16 = jnp.zeros((LANES,), jnp.float32)

    def body(r, _):
        for k in range(cols // LANES):
            ref[r, pl.ds(k * LANES, LANES)] = zero16
        return 0

    lax.fori_loop(0, rows, body, 0)


def _fill_vmem_1d(ref, n, value):
    val16 = jnp.full((LANES,), value, jnp.float32)

    def body(r, _):
        ref[pl.ds(r * LANES, LANES)] = val16
        return 0

    lax.fori_loop(0, n // LANES, body, 0)


def _sc_counts(dst3, nm3, sb3):
    @functools.partial(
        pl.kernel,
        out_type=(
            jax.ShapeDtypeStruct((NC * M_PAD,), jnp.float32),
            jax.ShapeDtypeStruct((NC * N_PAD,), jnp.float32),
            jax.ShapeDtypeStruct((NC * NP_PAD,), jnp.float32),
        ),
        mesh=_mesh(),
        scratch_types=[
            pltpu.VMEM((80, 128), jnp.int32),
            pltpu.VMEM((5, 128), jnp.int32),
            pltpu.VMEM((128,), jnp.float32),
            pltpu.VMEM((1280,), jnp.float32),
            pltpu.VMEM_SHARED((M_PAD,), jnp.float32),
            pltpu.VMEM_SHARED((N_PAD,), jnp.float32),
            pltpu.VMEM_SHARED((NP_PAD,), jnp.float32),
        ],
    )
    def k(dst_h, nm_h, sb_h, deg_o, cnm_o, csb_o, idx_v, idx5_v, ones_v,
          zero_v, deg_a, cnm_a, csb_a):
        cid = lax.axis_index("c")
        sid = lax.axis_index("s")
        wid = cid * NS + sid
        _fill_vmem_1d(ones_v, 128, 1.0)
        _fill_vmem_1d(zero_v, 1280, 0.0)
        pltpu.sync_copy(zero_v, deg_a.at[pl.ds(_al8(sid * 1280), 1280)])
        pltpu.sync_copy(zero_v.at[pl.ds(0, 640)],
                        cnm_a.at[pl.ds(_al8(sid * 640), 640)])

        @pl.when(sid < 5)
        def _():
            pltpu.sync_copy(zero_v.at[pl.ds(0, 128)],
                            csb_a.at[pl.ds(_al8(sid * 128), 128)])

        plsc.subcore_barrier()
        pltpu.sync_copy(dst_h.at[wid], idx_v)

        def dbody(j, _):
            pltpu.sync_copy(ones_v, deg_a.at[idx_v.at[j]], add=True)
            return 0

        lax.fori_loop(0, 80, dbody, 0)
        pltpu.sync_copy(nm_h.at[wid], idx5_v)

        def nbody(j, _):
            pltpu.sync_copy(ones_v, cnm_a.at[idx5_v.at[j]], add=True)
            return 0

        lax.fori_loop(0, 5, nbody, 0)
        pltpu.sync_copy(sb_h.at[wid], idx5_v)

        def sbody(j, _):
            pltpu.sync_copy(ones_v, csb_a.at[idx5_v.at[j]], add=True)
            return 0

        lax.fori_loop(0, 5, sbody, 0)
        plsc.subcore_barrier()
        pltpu.sync_copy(deg_a.at[pl.ds(_al8(sid * 1280), 1280)],
                        deg_o.at[pl.ds(_al8(cid * M_PAD + sid * 1280), 1280)])
        pltpu.sync_copy(cnm_a.at[pl.ds(_al8(sid * 640), 640)],
                        cnm_o.at[pl.ds(_al8(cid * N_PAD + sid * 640), 640)])

        @pl.when(sid < 5)
        def _():
            pltpu.sync_copy(
                csb_a.at[pl.ds(_al8(sid * 128), 128)],
                csb_o.at[pl.ds(_al8(cid * NP_PAD + sid * 128), 128)])

    deg_f, cnm_f, csb_f = k(dst3, nm3, sb3)
    return (deg_f.reshape(NC, M_PAD), cnm_f.reshape(NC, N_PAD),
            csb_f.reshape(NC, NP_PAD))


def _sc_gather(tab, idx3):
    rpw = idx3.shape[1]

    @functools.partial(
        pl.kernel,
        out_type=jax.ShapeDtypeStruct((NW * rpw * 128, 128), jnp.float32),
        mesh=_mesh(),
        scratch_types=[
            pltpu.VMEM((rpw, 128), jnp.int32),
            pltpu.VMEM((128, 128), jnp.float32),
            pltpu.VMEM((128, 128), jnp.float32),
            pltpu.SemaphoreType.DMA,
            pltpu.SemaphoreType.DMA,
        ],
    )
    def k(tab_h, idx_h, out_h, idx_v, rows_a, rows_b, sem_a, sem_b):
        cid = lax.axis_index("c")
        sid = lax.axis_index("s")
        wid = cid * NS + sid
        base = wid * rpw
        pltpu.sync_copy(idx_h.at[wid], idx_v)
        bufs = (rows_a, rows_b)
        sems = (sem_a, sem_b)
        descs = [None, None]
        descs[0] = pltpu.async_copy(tab_h.at[idx_v.at[0]], rows_a, sem_a)
        for j in range(rpw):
            cur = j % 2
            nxt = (j + 1) % 2
            if j + 1 < rpw:
                descs[nxt] = pltpu.async_copy(tab_h.at[idx_v.at[j + 1]],
                                              bufs[nxt], sems[nxt])
            descs[cur].wait()
            pltpu.sync_copy(bufs[cur],
                            out_h.at[pl.ds(_al8((base + j) * 128), 128)])

    return k(tab, idx3)


HALF_M = M_PAD // 2
ACC_M = 10496
IBLK = 32


def _sc_edge_segsum(hn, srcE, dstE):
    rpt = E_PAD // 128 // NS
    nblk = rpt // IBLK

    @functools.partial(
        pl.kernel,
        out_type=jax.ShapeDtypeStruct((M_PAD, H), jnp.float32),
        mesh=_mesh(),
        scratch_types=[
            pltpu.VMEM((IBLK, 128), jnp.int32),
            pltpu.VMEM((IBLK, 128), jnp.int32),
            pltpu.VMEM((128, 128), jnp.float32),
            pltpu.VMEM((128, 128), jnp.float32),
            pltpu.VMEM_SHARED((ACC_M, 128), jnp.float32),
            pltpu.SemaphoreType.DMA,
            pltpu.SemaphoreType.DMA,
        ],
    )
    def k(tab_h, src_h, dst_h, out_h, src_v, dst_v, rows_a, rows_b, acc,
          sem_a, sem_b):
        zrow_v = rows_a
        cid = lax.axis_index("c")
        sid = lax.axis_index("s")
        _zero_vmem_2d(zrow_v, 128, 128)

        def zbody(t, _):
            pltpu.sync_copy(zrow_v,
                            acc.at[pl.ds(_al8((sid * 5 + t) * 128), 128)])
            return 0

        lax.fori_loop(0, 5, zbody, 0)

        @pl.when(sid < 2)
        def _():
            pltpu.sync_copy(zrow_v, acc.at[pl.ds(_al8((80 + sid) * 128), 128)])

        base = cid * HALF_M
        plsc.subcore_barrier()

        def blk(bi, _):
            pltpu.sync_copy(
                src_h.at[sid, pl.ds(_al8(bi * IBLK), IBLK)], src_v)
            pltpu.sync_copy(
                dst_h.at[sid, pl.ds(_al8(bi * IBLK), IBLK)], dst_v)

            def tbody(j, _):
                for kk in range(8):
                    v = dst_v[j, pl.ds(kk * LANES, LANES)] - base
                    v = jnp.where((v < 0) | (v >= HALF_M), HALF_M, v)
                    dst_v[j, pl.ds(kk * LANES, LANES)] = v
                return 0

            lax.fori_loop(0, IBLK, tbody, 0)

            bufs = (rows_a, rows_b)
            sems = (sem_a, sem_b)
            descs = [None, None]
            descs[0] = pltpu.async_copy(tab_h.at[src_v.at[0]], rows_a, sem_a)
            for j in range(IBLK):
                cur = j % 2
                nxt = (j + 1) % 2
                if j + 1 < IBLK:
                    descs[nxt] = pltpu.async_copy(tab_h.at[src_v.at[j + 1]],
                                                  bufs[nxt], sems[nxt])
                descs[cur].wait()
                pltpu.sync_copy(bufs[cur], acc.at[dst_v.at[j]], add=True)
            return 0

        lax.fori_loop(0, nblk, blk, 0)
        plsc.subcore_barrier()

        def obody(t, _):
            r0 = _al8((sid * 5 + t) * 128)
            pltpu.sync_copy(acc.at[pl.ds(r0, 128)],
                            out_h.at[pl.ds(_al8(cid * HALF_M + r0), 128)])
            return 0

        lax.fori_loop(0, 5, obody, 0)

    return k(hn, srcE, dstE)


def _sc_patch_segsum(data, sb3):
    @functools.partial(
        pl.kernel,
        out_type=jax.ShapeDtypeStruct((NC, NP_PAD, 128), jnp.float32),
        mesh=_mesh(),
        scratch_types=[
            pltpu.VMEM((5, 128), jnp.int32),
            pltpu.VMEM((128, 128), jnp.float32),
            pltpu.VMEM((128, 128), jnp.float32),
            pltpu.VMEM_SHARED((NP_PAD, 128), jnp.float32),
        ],
    )
    def k(data_h, sb_h, out_h, idx_v, data_v, zrow_v, acc):
        cid = lax.axis_index("c")
        sid = lax.axis_index("s")
        wid = cid * NS + sid
        _zero_vmem_2d(zrow_v, 128, 128)
        pltpu.sync_copy(zrow_v.at[pl.ds(0, 40)],
                        acc.at[pl.ds(_al8(sid * 40), 40)])
        pltpu.sync_copy(sb_h.at[wid], idx_v)
        plsc.subcore_barrier()

        def body(j, _):
            pltpu.sync_copy(
                data_h.at[pl.ds(_al8((wid * 5 + j) * 128), 128)], data_v)
            pltpu.sync_copy(data_v, acc.at[idx_v.at[j]], add=True)
            return 0

        lax.fori_loop(0, 5, body, 0)
        plsc.subcore_barrier()
        pltpu.sync_copy(acc.at[pl.ds(_al8(sid * 40), 40)],
                        out_h.at[cid, pl.ds(_al8(sid * 40), 40)])

    return k(data, sb3)


def _sc_node_segsum(data, nm3):
    @functools.partial(
        pl.kernel,
        out_type=jax.ShapeDtypeStruct((NC, N_PAD, H), jnp.float32),
        mesh=_mesh(),
        scratch_types=[
            pltpu.VMEM((5, 128), jnp.int32),
            pltpu.VMEM((128, 128), jnp.float32),
            pltpu.VMEM((128, 128), jnp.float32),
            pltpu.VMEM_SHARED((N_PAD, H), jnp.float32),
        ],
    )
    def k(data_h, nm_h, out_h, idx_v, data_v, zrow_v, acc):
        cid = lax.axis_index("c")
        sid = lax.axis_index("s")
        wid = cid * NS + sid
        _zero_vmem_2d(zrow_v, 128, 128)

        def zbody(t, _):
            pltpu.sync_copy(zrow_v,
                            acc.at[pl.ds(_al8((sid * 5 + t) * 128), 128)])
            return 0

        lax.fori_loop(0, 5, zbody, 0)
        pltpu.sync_copy(nm_h.at[wid], idx_v)
        plsc.subcore_barrier()

        def body(j, _):
            pltpu.sync_copy(
                data_h.at[pl.ds(_al8((wid * 5 + j) * 128), 128)], data_v)
            pltpu.sync_copy(data_v, acc.at[idx_v.at[j]], add=True)
            return 0

        lax.fori_loop(0, 5, body, 0)
        plsc.subcore_barrier()

        def obody(t, _):
            r0 = _al8((sid * 5 + t) * 128)
            pltpu.sync_copy(acc.at[pl.ds(r0, 128)],
                            out_h.at[cid, pl.ds(r0, 128)])
            return 0

        lax.fori_loop(0, 5, obody, 0)

    return k(data, nm3)


def _tc_pre(x, W, b):
    def body(x_r, w_r, b_r, o_r):
        o_r[...] = jnp.maximum(
            jax.lax.dot_general(x_r[...], w_r[...], (((1,), (0,)), ((), ())),
                                precision=_HI) + b_r[...], 0.0)

    return pl.pallas_call(
        body,
        out_shape=jax.ShapeDtypeStruct((N, H), jnp.float32),
        grid=(5,),
        in_specs=[
            pl.BlockSpec((2000, H), lambda i: (i, 0)),
            pl.BlockSpec((H, H), lambda i: (0, 0)),
            pl.BlockSpec((1, H), lambda i: (0, 0)),
        ],
        out_specs=pl.BlockSpec((2000, H), lambda i: (i, 0)),
    )(x, W, b)


def _tc_gcn_lin(xs, W, b, degT):
    def body(x_r, w_r, b_r, d_r, u_r, hn_r):
        deg = d_r[:, 0:1] + d_r[:, 1:2] + 1.0
        isd = lax.rsqrt(deg)
        hh = jax.lax.dot_general(x_r[...], w_r[...], (((1,), (0,)), ((), ())),
                                 precision=_HI) + b_r[...]
        u_r[...] = hh / deg
        hn_r[...] = hh * isd

    return pl.pallas_call(
        body,
        out_shape=(
            jax.ShapeDtypeStruct((M_PAD, H), jnp.float32),
            jax.ShapeDtypeStruct((M_PAD, H), jnp.float32),
        ),
        grid=(10,),
        in_specs=[
            pl.BlockSpec((2048, H), lambda i: (i, 0)),
            pl.BlockSpec((H, H), lambda i: (0, 0)),
            pl.BlockSpec((1, H), lambda i: (0, 0)),
            pl.BlockSpec((2048, 2), lambda i: (i, 0)),
        ],
        out_specs=(
            pl.BlockSpec((2048, H), lambda i: (i, 0)),
            pl.BlockSpec((2048, H), lambda i: (i, 0)),
        ),
    )(xs, W, b, degT)


def _tc_gcn_combine(tmp, u, degT):
    def body(t_r, u_r, d_r, o_r):
        deg = d_r[:, 0:1] + d_r[:, 1:2] + 1.0
        isd = lax.rsqrt(deg)
        o_r[...] = jnp.maximum(t_r[...] * isd + u_r[...], 0.0)

    return pl.pallas_call(
        body,
        out_shape=jax.ShapeDtypeStruct((M_PAD, H), jnp.float32),
        grid=(10,),
        in_specs=[
            pl.BlockSpec((2048, H), lambda i: (i, 0)),
            pl.BlockSpec((2048, H), lambda i: (i, 0)),
            pl.BlockSpec((2048, 2), lambda i: (i, 0)),
        ],
        out_specs=pl.BlockSpec((2048, H), lambda i: (i, 0)),
    )(tmp, u, degT)


def _tc_patch_mlp(p0, p1, cntT, W, b):
    def body(p0_r, p1_r, c_r, w_r, b_r, o_r):
        cnt = jnp.maximum(c_r[:, 0:1] + c_r[:, 1:2], 1.0)
        pm = (p0_r[...] + p1_r[...]) / cnt
        o_r[...] = jnp.maximum(
            jax.lax.dot_general(pm, w_r[...], (((1,), (0,)), ((), ())),
                                precision=_HI) + b_r[...], 0.0)

    return pl.pallas_call(
        body,
        out_shape=jax.ShapeDtypeStruct((NP_PAD, H), jnp.float32),
        in_specs=[
            pl.BlockSpec((NP_PAD, H), lambda: (0, 0)),
            pl.BlockSpec((NP_PAD, H), lambda: (0, 0)),
            pl.BlockSpec((NP_PAD, 2), lambda: (0, 0)),
            pl.BlockSpec((H, H), lambda: (0, 0)),
            pl.BlockSpec((1, H), lambda: (0, 0)),
        ],
        out_specs=pl.BlockSpec((NP_PAD, H), lambda: (0, 0)),
    )(p0, p1, cntT, W, b)


def _tc_add(xs1, g1):
    def body(a_r, b_r, o_r):
        o_r[...] = a_r[...] + b_r[...]

    return pl.pallas_call(
        body,
        out_shape=jax.ShapeDtypeStruct((M_PAD, H), jnp.float32),
        grid=(10,),
        in_specs=[
            pl.BlockSpec((2048, H), lambda i: (i, 0)),
            pl.BlockSpec((2048, H), lambda i: (i, 0)),
        ],
        out_specs=pl.BlockSpec((2048, H), lambda i: (i, 0)),
    )(xs1, g1)


def _tc_node_mean(s0, s1, cntT):
    def body(a_r, b_r, c_r, o_r):
        cnt = jnp.maximum(c_r[:, 0:1] + c_r[:, 1:2], 1.0)
        o_r[...] = (a_r[...] + b_r[...]) / cnt

    return pl.pallas_call(
        body,
        out_shape=jax.ShapeDtypeStruct((N_PAD, H), jnp.float32),
        grid=(5,),
        in_specs=[
            pl.BlockSpec((2048, H), lambda i: (i, 0)),
            pl.BlockSpec((2048, H), lambda i: (i, 0)),
            pl.BlockSpec((2048, 2), lambda i: (i, 0)),
        ],
        out_specs=pl.BlockSpec((2048, H), lambda i: (i, 0)),
    )(s0, s1, cntT)


def _ln(z):
    mu = z.mean(-1, keepdims=True)
    var = z.var(-1, keepdims=True)
    return (z - mu) / jnp.sqrt(var + 1e-5)


def _tc_mixer_head(p0, p1, cntT, patch_pe, W_pe1, b_pe1, W_pe2, b_pe2,
                   Wt1T, bt1c, Wt2T, bt2c, Wc1, bc1r, Wc2, bc2r, mask,
                   W_head, b_head):
    def body(p0_r, p1_r, c_r, pe_r, wp1_r, bp1_r, wp2_r, bp2_r, wt1_r, bt1_r,
             wt2_r, bt2_r, wc1_r, bc1_r, wc2_r, bc2_r, m_r, wh_r, bh_r, o_r):
        cnt = jnp.maximum(c_r[:, 0:1] + c_r[:, 1:2], 1.0)
        sub = (p0_r[...] + p1_r[...]) / cnt
        pe = jax.lax.dot_general(
            jnp.maximum(jax.lax.dot_general(pe_r[...], wp1_r[...],
                                            (((1,), (0,)), ((), ())),
                                            precision=_HI) + bp1_r[...], 0.0),
            wp2_r[...], (((1,), (0,)), ((), ())), precision=_HI) + bp2_r[...]
        z = sub + pe
        wt1 = wt1_r[...]
        bt1 = bt1_r[...]
        wt2 = wt2_r[...]
        bt2 = bt2_r[...]
        wc1 = wc1_r[...]
        bc1 = bc1_r[...]
        wc2 = wc2_r[...]
        bc2 = bc2_r[...]
        for l in range(L):
            y = _ln(z)
            wt1l = wt1[l * P:(l + 1) * P, :]
            wt2l = wt2[l * P:(l + 1) * P, :]
            bt1l = bt1[l * P:(l + 1) * P, :]
            bt2l = bt2[l * P:(l + 1) * P, :]
            touts = []
            for b in range(B):
                yb = y[b * P:(b + 1) * P, :]
                a = jax.lax.dot_general(wt1l, yb, (((1,), (0,)), ((), ())),
                                        precision=_HI) + bt1l
                a = jax.nn.gelu(a)
                a = jax.lax.dot_general(wt2l, a, (((1,), (0,)), ((), ())),
                                        precision=_HI) + bt2l
                touts.append(a)
            z = z + jnp.concatenate(touts, axis=0)
            y = _ln(z)
            cmid = jax.nn.gelu(
                jax.lax.dot_general(y, wc1[l * H:(l + 1) * H, :],
                                    (((1,), (0,)), ((), ())),
                                    precision=_HI) + bc1[l:l + 1, :])
            z = z + jax.lax.dot_general(cmid, wc2[l * 2 * H:(l + 1) * 2 * H, :],
                                        (((1,), (0,)), ((), ())),
                                        precision=_HI) + bc2[l:l + 1, :]
        m = m_r[...]
        gs = []
        for b in range(B):
            mb = m[b:b + 1, :]
            zb = z[b * P:(b + 1) * P, :]
            gb = jax.lax.dot_general(mb, zb, (((1,), (0,)), ((), ())),
                                     precision=_HI)
            gs.append(gb / jnp.sum(mb, keepdims=True))
        g = jnp.concatenate(gs, axis=0)
        o_r[...] = jax.lax.dot_general(g, wh_r[...], (((1,), (0,)), ((), ())),
                                       precision=_HI) + bh_r[...]

    return pl.pallas_call(
        body,
        out_shape=jax.ShapeDtypeStruct((B, C), jnp.float32),
        in_specs=[
            pl.BlockSpec((NP, H), lambda: (0, 0)),
            pl.BlockSpec((NP, H), lambda: (0, 0)),
            pl.BlockSpec((NP, 2), lambda: (0, 0)),
            pl.BlockSpec((NP, PE), lambda: (0, 0)),
            pl.BlockSpec((PE, H), lambda: (0, 0)),
            pl.BlockSpec((1, H), lambda: (0, 0)),
            pl.BlockSpec((H, H), lambda: (0, 0)),
            pl.BlockSpec((1, H), lambda: (0, 0)),
            pl.BlockSpec((L * P, P), lambda: (0, 0)),
            pl.BlockSpec((L * P, 1), lambda: (0, 0)),
            pl.BlockSpec((L * P, P), lambda: (0, 0)),
            pl.BlockSpec((L * P, 1), lambda: (0, 0)),
            pl.BlockSpec((L * H, 2 * H), lambda: (0, 0)),
            pl.BlockSpec((L, 2 * H), lambda: (0, 0)),
            pl.BlockSpec((L * 2 * H, H), lambda: (0, 0)),
            pl.BlockSpec((L, H), lambda: (0, 0)),
            pl.BlockSpec((B, P), lambda: (0, 0)),
            pl.BlockSpec((H, C), lambda: (0, 0)),
            pl.BlockSpec((1, C), lambda: (0, 0)),
        ],
        out_specs=pl.BlockSpec((B, C), lambda: (0, 0)),
    )(p0, p1, cntT, patch_pe, W_pe1, b_pe1, W_pe2, b_pe2,
      Wt1T, bt1c, Wt2T, bt2c, Wc1, bc1r, Wc2, bc2r, mask, W_head, b_head)


def _pad_idx(a, total, fill, lead):
    a = a.astype(jnp.int32)
    return jnp.concatenate(
        [a, jnp.full((total - a.shape[0],), fill, jnp.int32)]
    ).reshape(lead, -1, 128)


def kernel(x, edge_index, nodes_mapper, subgraphs_batch, patch_pe, mask,
           W_pre, b_pre, W_gcn0, b_gcn0, W_mid, b_mid, W_gcn1, b_gcn1,
           W_pe1, b_pe1, W_pe2, b_pe2, Wt1, bt1, Wt2, bt2, Wc1, bc1, Wc2, bc2,
           W_head, b_head):
    srcE = _pad_idx(edge_index[0], E_PAD, 0, NS)
    dstE = _pad_idx(edge_index[1], E_PAD, M, NS)
    dst3 = _pad_idx(edge_index[1], E_PAD, M, NW)
    nmG = _pad_idx(nodes_mapper, M_PAD, 0, NW)
    nmS = _pad_idx(nodes_mapper, M_PAD, N, NW)
    sbR = _pad_idx(subgraphs_batch, M_PAD, NP, NW)

    b_pre2 = b_pre.reshape(1, H)
    b_gcn0r = b_gcn0.reshape(1, H)
    b_gcn1r = b_gcn1.reshape(1, H)
    b_midr = b_mid.reshape(1, H)
    b_pe1r = b_pe1.reshape(1, H)
    b_pe2r = b_pe2.reshape(1, H)
    Wt1T = jnp.swapaxes(Wt1, 1, 2).reshape(L * P, P)
    Wt2T = jnp.swapaxes(Wt2, 1, 2).reshape(L * P, P)
    bt1c = bt1.reshape(L * P, 1)
    bt2c = bt2.reshape(L * P, 1)
    Wc1r = Wc1.reshape(L * H, 2 * H)
    Wc2r = Wc2.reshape(L * 2 * H, H)
    b_headr = b_head.reshape(1, C)

    h = _tc_pre(x, W_pre, b_pre2)
    deg_p, cnm_p, csb_p = _sc_counts(dst3, nmS, sbR)
    degT = jnp.swapaxes(deg_p, 0, 1)
    cnmT = jnp.swapaxes(cnm_p, 0, 1)
    csbT = jnp.swapaxes(csb_p, 0, 1)
    xs0 = _sc_gather(h, nmG)
    u0, hn0 = _tc_gcn_lin(xs0, W_gcn0, b_gcn0r, degT)
    tmp0 = _sc_edge_segsum(hn0, srcE, dstE)
    xs1 = _tc_gcn_combine(tmp0, u0, degT)
    pool1 = _sc_patch_segsum(xs1, sbR)
    q = _tc_patch_mlp(pool1[0], pool1[1], csbT, W_mid, b_midr)
    g1 = _sc_gather(q, sbR)
    xs2 = _tc_add(xs1, g1)
    s = _sc_node_segsum(xs2, nmS)
    meantab = _tc_node_mean(s[0], s[1], cnmT)
    xs4 = _sc_gather(meantab, nmS)
    u1, hn1 = _tc_gcn_lin(xs4, W_gcn1, b_gcn1r, degT)
    tmp1 = _sc_edge_segsum(hn1, srcE, dstE)
    xs5 = _tc_gcn_combine(tmp1, u1, degT)
    pool2 = _sc_patch_segsum(xs5, sbR)
    out = _tc_mixer_head(pool2[0, :NP], pool2[1, :NP], csbT[:NP], patch_pe,
                         W_pe1, b_pe1r, W_pe2, b_pe2r, Wt1T, bt1c, Wt2T, bt2c,
                         Wc1r, bc1, Wc2r, bc2, mask, W_head, b_headr)
    return out
